# baseline (speedup 1.0000x reference)
import functools

import jax
import jax.numpy as jnp
import numpy as np
from jax.experimental import pallas as pl
from jax.experimental.pallas import tpu as pltpu

_SET_Q = 4
_BT = 4


def _body(idx_ref, obs_ref, ctx_ref, qry_ref, *, ctx_len):
    ctx_ref[...] = obs_ref[:, :ctx_len, :]
    for i in range(_SET_Q):
        qry_ref[:, i, :] = obs_ref[:, idx_ref[i], :]


def kernel(obs, set_q_idx):
    b, s, d = obs.shape
    ctx_len = s - _SET_Q
    base_idx = np.random.default_rng(0).choice(
        s, size=_SET_Q, replace=False).astype(np.int32)
    qidx = jnp.asarray(base_idx) + (
        jnp.asarray(set_q_idx, dtype=jnp.int32) - _SET_Q)
    grid = (b // _BT,)
    context, query = pl.pallas_call(
        functools.partial(_body, ctx_len=ctx_len),
        grid=grid,
        in_specs=[
            pl.BlockSpec(memory_space=pltpu.SMEM),
            pl.BlockSpec((_BT, s, d), lambda i: (i, 0, 0)),
        ],
        out_specs=(
            pl.BlockSpec((_BT, ctx_len, d), lambda i: (i, 0, 0)),
            pl.BlockSpec((_BT, _SET_Q, d), lambda i: (i, 0, 0)),
        ),
        out_shape=(
            jax.ShapeDtypeStruct((b, ctx_len, d), obs.dtype),
            jax.ShapeDtypeStruct((b, _SET_Q, d), obs.dtype),
        ),
    )(qidx, obs)
    return (context, query)


# speedup vs baseline: 1.0118x; 1.0118x over previous
import functools

import jax
import jax.numpy as jnp
import numpy as np
from jax.experimental import pallas as pl
from jax.experimental.pallas import tpu as pltpu

_SET_Q = 4
_BT = 8


def _body(idx_ref, obs_ref, ctx_ref, qry_ref, *, ctx_len):
    ctx_ref[...] = obs_ref[:, :ctx_len, :]
    for i in range(_SET_Q):
        qry_ref[:, i, :] = obs_ref[:, idx_ref[i], :]


def kernel(obs, set_q_idx):
    b, s, d = obs.shape
    ctx_len = s - _SET_Q
    base_idx = np.random.default_rng(0).choice(
        s, size=_SET_Q, replace=False).astype(np.int32)
    qidx = jnp.asarray(base_idx) + (
        jnp.asarray(set_q_idx, dtype=jnp.int32) - _SET_Q)
    grid = (b // _BT,)
    context, query = pl.pallas_call(
        functools.partial(_body, ctx_len=ctx_len),
        grid=grid,
        in_specs=[
            pl.BlockSpec(memory_space=pltpu.SMEM),
            pl.BlockSpec((_BT, s, d), lambda i: (i, 0, 0)),
        ],
        out_specs=(
            pl.BlockSpec((_BT, ctx_len, d), lambda i: (i, 0, 0)),
            pl.BlockSpec((_BT, _SET_Q, d), lambda i: (i, 0, 0)),
        ),
        out_shape=(
            jax.ShapeDtypeStruct((b, ctx_len, d), obs.dtype),
            jax.ShapeDtypeStruct((b, _SET_Q, d), obs.dtype),
        ),
        compiler_params=pltpu.CompilerParams(
            dimension_semantics=("parallel",),
        ),
    )(qidx, obs)
    return (context, query)
